# baseline (speedup 1.0000x reference)
import functools

import jax
import jax.numpy as jnp
from jax import lax
from jax.experimental import pallas as pl
from jax.experimental.pallas import tpu as pltpu
from jax.experimental.pallas import tpu_sc as plsc

N = 10000
E = 320000
D = 128
NC = 2
NS = 16
CH = 80
EPT = E // (NC * NS)
KC = EPT // CH
RPW = 624


def _rowwise(s, copy):
    copy(pl.ds(s * RPW, RPW))

    @pl.when(s == NS - 1)
    def _():
        copy(pl.ds(NS * RPW, N - NS * RPW))


def _agg_body(h, src2d, dst2d, zeros, out0, out1,
              acc, src_v, dst_v, rows0, rows1, sem0, sem1):
    c = lax.axis_index("c")
    s = lax.axis_index("s")

    _rowwise(s, lambda r: pltpu.sync_copy(zeros.at[r], acc.at[r]))

    tile = c * NS + s
    pltpu.sync_copy(src2d.at[tile], src_v)
    pltpu.sync_copy(dst2d.at[tile], dst_v)

    plsc.subcore_barrier()

    sems = (sem0, sem1)
    rows = (rows0, rows1)

    def gather(j, b):
        return pltpu.make_async_copy(
            h.at[src_v.at[pl.ds(j * CH, CH)]], rows[b], sems[b])

    def scatter(j, b):
        gather(j, b).wait()
        pltpu.sync_copy(rows[b], acc.at[dst_v.at[j]], add=True)

    gather(0, 0).start()

    def step(i, carry):
        j = 2 * i
        gather(j + 1, 1).start()
        scatter(j, 0)

        @pl.when(j + 2 < KC)
        def _():
            gather(j + 2, 0).start()

        scatter(j + 1, 1)
        return carry

    lax.fori_loop(0, KC // 2, step, 0)
    if KC % 2:
        scatter(KC - 1, 0)

    plsc.subcore_barrier()

    @pl.when(c == 0)
    def _():
        _rowwise(s, lambda r: pltpu.sync_copy(acc.at[r], out0.at[r]))

    @pl.when(c == 1)
    def _():
        _rowwise(s, lambda r: pltpu.sync_copy(acc.at[r], out1.at[r]))


_agg_plain = pl.kernel(
    _agg_body,
    out_type=[jax.ShapeDtypeStruct((N, D), jnp.float32),
              jax.ShapeDtypeStruct((N, D), jnp.float32)],
    mesh=plsc.VectorSubcoreMesh(core_axis_name="c", subcore_axis_name="s"),
    compiler_params=pltpu.CompilerParams(skip_device_barrier=True),
    scratch_types=[
        pltpu.VMEM_SHARED((N, D), jnp.float32),
        pltpu.VMEM((EPT,), jnp.int32),
        pltpu.VMEM((KC, CH), jnp.int32),
        pltpu.VMEM((CH, D), jnp.float32),
        pltpu.VMEM((CH, D), jnp.float32),
        pltpu.SemaphoreType.DMA,
        pltpu.SemaphoreType.DMA,
    ])


_L = 16
_NV = EPT // _L
_NPAD = 10112
_CB = 640


def _count_body(dst1d, zerosp, cnt0, cnt1,
                cnt_all, dst_flat, cnt_local, slab, out_stage):
    c = lax.axis_index("c")
    s = lax.axis_index("s")

    pltpu.sync_copy(dst1d.at[c * NS + s], dst_flat)
    pltpu.sync_copy(zerosp, cnt_local)
    ones_v = jnp.ones((_L,), jnp.float32)

    def cstep(i, carry):
        idx = dst_flat[pl.ds(i * _L, _L)]
        plsc.addupdate_scatter(cnt_local, [idx], ones_v)
        return carry

    lax.fori_loop(0, _NV, cstep, 0)

    pltpu.sync_copy(cnt_local, cnt_all.at[s])
    plsc.subcore_barrier()

    def reduce_range(ncols):
        pltpu.sync_copy(cnt_all.at[:, pl.ds(s * _CB, ncols)],
                        slab.at[:, pl.ds(0, ncols)])

        col0 = jnp.zeros((_L,), jnp.int32)
        lanes = lax.iota(jnp.int32, _L)

        def rstep(b, carry):
            tot = slab[0, pl.ds(b * _L, _L)]
            for t in range(1, NS):
                tot = tot + slab[t, pl.ds(b * _L, _L)]
            plsc.store_scatter(out_stage, [b * _L + lanes, col0], tot)
            return carry

        lax.fori_loop(0, ncols // _L, rstep, 0)

    @pl.when(s < NS - 1)
    def _():
        reduce_range(_CB)

    @pl.when(s == NS - 1)
    def _():
        reduce_range(_NPAD - (NS - 1) * _CB)

    def write(o):
        @pl.when(s < NS - 1)
        def _():
            pltpu.sync_copy(out_stage.at[pl.ds(0, _CB)],
                            o.at[pl.ds(s * _CB, _CB)])

        @pl.when(s == NS - 1)
        def _():
            pltpu.sync_copy(out_stage.at[pl.ds(0, N - (NS - 1) * _CB)],
                            o.at[pl.ds((NS - 1) * _CB, N - (NS - 1) * _CB)])

    @pl.when(c == 0)
    def _():
        write(cnt0)

    @pl.when(c == 1)
    def _():
        write(cnt1)


_count = pl.kernel(
    _count_body,
    out_type=[jax.ShapeDtypeStruct((N, _L), jnp.float32),
              jax.ShapeDtypeStruct((N, _L), jnp.float32)],
    mesh=plsc.VectorSubcoreMesh(core_axis_name="c", subcore_axis_name="s"),
    compiler_params=pltpu.CompilerParams(needs_layout_passes=False,
                                         skip_device_barrier=True),
    scratch_types=[
        pltpu.VMEM_SHARED((NS, _NPAD), jnp.float32),
        pltpu.VMEM((EPT,), jnp.int32),
        pltpu.VMEM((_NPAD,), jnp.float32),
        pltpu.VMEM((NS, _CB), jnp.float32),
        pltpu.VMEM((_CB, _L), jnp.float32),
    ])


_BLK = 1000
_row_spec = pl.BlockSpec((_BLK, D), lambda i: (i, 0))
_cnt_spec = pl.BlockSpec((_BLK, _L), lambda i: (i, 0))
_w_spec = pl.BlockSpec((D, D), lambda i: (0, 0))
_b_spec = pl.BlockSpec((1, D), lambda i: (0, 0))


def _densew_body(h, wr, b, o):
    o[...] = (jnp.dot(h[...], wr[...], preferred_element_type=jnp.float32)
              + b[...])


def _densew(h, wrt, b2d):
    return pl.pallas_call(
        _densew_body,
        grid=(N // _BLK,),
        in_specs=[_row_spec, _w_spec, _b_spec],
        out_specs=_row_spec,
        out_shape=jax.ShapeDtypeStruct((N, D), jnp.float32),
    )(h, wrt, b2d)


def _densec_body(relu, p0, p1, c0, c1, hr, wl, o):
    cnt = c0[:, 0:1] + c1[:, 0:1]
    inv = 1.0 / jnp.maximum(cnt, 1.0)
    agg = (p0[...] + p1[...]) * inv
    out = jnp.dot(agg, wl[...], preferred_element_type=jnp.float32) + hr[...]
    o[...] = jnp.maximum(out, 0.0) if relu else out


def _densec(p0, p1, c0, c1, hr, wlt, relu):
    return pl.pallas_call(
        functools.partial(_densec_body, relu),
        grid=(N // _BLK,),
        in_specs=[_row_spec, _row_spec, _cnt_spec, _cnt_spec, _row_spec,
                  _w_spec],
        out_specs=_row_spec,
        out_shape=jax.ShapeDtypeStruct((N, D), jnp.float32),
    )(p0, p1, c0, c1, hr, wlt)


def kernel(x, e, Wl1, Wr1, b1, Wl2, Wr2, b2, Wl3, Wr3, b3):
    src2d = e[0].reshape(NC * NS, EPT)
    dst2d = e[1].reshape(NC * NS, KC, CH)
    dst1d = e[1].reshape(NC * NS, EPT)
    zeros = jnp.zeros((N, D), jnp.float32)
    zerosp = jnp.zeros((_NPAD,), jnp.float32)

    hr1 = _densew(x, Wr1.T, b1.reshape(1, D))
    c0, c1 = _count(dst1d, zerosp)
    p0, p1 = _agg_plain(x, src2d, dst2d, zeros)
    h1 = _densec(p0, p1, c0, c1, hr1, Wl1.T, True)
    hr2 = _densew(h1, Wr2.T, b2.reshape(1, D))
    p0, p1 = _agg_plain(h1, src2d, dst2d, zeros)
    h2 = _densec(p0, p1, c0, c1, hr2, Wl2.T, True)
    hr3 = _densew(h2, Wr3.T, b3.reshape(1, D))
    p0, p1 = _agg_plain(h2, src2d, dst2d, zeros)
    return _densec(p0, p1, c0, c1, hr3, Wl3.T, False)


# speedup vs baseline: 1.0197x; 1.0197x over previous
import functools

import jax
import jax.numpy as jnp
from jax import lax
from jax.experimental import pallas as pl
from jax.experimental.pallas import tpu as pltpu
from jax.experimental.pallas import tpu_sc as plsc

N = 10000
E = 320000
D = 128
NC = 2
NS = 16
CH = 80
EPT = E // (NC * NS)
KC = EPT // CH
RPW = 624


def _rowwise(s, copy):
    copy(pl.ds(s * RPW, RPW))

    @pl.when(s == NS - 1)
    def _():
        copy(pl.ds(NS * RPW, N - NS * RPW))


def _agg_body(h, src2d, dst2d, zeros, out0, out1,
              acc, src_v, dst_v, rows0, rows1, sem0, sem1):
    c = lax.axis_index("c")
    s = lax.axis_index("s")
    sems = (sem0, sem1)
    rows = (rows0, rows1)

    def gather(j, b):
        return pltpu.make_async_copy(
            h.at[src_v.at[pl.ds(j * CH, CH)]], rows[b], sems[b])

    def zcopy(r):
        return pltpu.make_async_copy(zeros.at[r], acc.at[r], sem1)

    rmain = pl.ds(s * RPW, RPW)
    rtail = pl.ds(NS * RPW, N - NS * RPW)
    zcopy(rmain).start()

    @pl.when(s == NS - 1)
    def _():
        zcopy(rtail).start()

    tile = c * NS + s
    pltpu.sync_copy(src2d.at[tile], src_v)
    gather(0, 0).start()
    pltpu.sync_copy(dst2d.at[tile], dst_v)

    zcopy(rmain).wait()

    @pl.when(s == NS - 1)
    def _():
        zcopy(rtail).wait()

    plsc.subcore_barrier()

    def scatter(j, b):
        gather(j, b).wait()
        pltpu.sync_copy(rows[b], acc.at[dst_v.at[j]], add=True)


    def step(i, carry):
        j = 2 * i
        gather(j + 1, 1).start()
        scatter(j, 0)

        @pl.when(j + 2 < KC)
        def _():
            gather(j + 2, 0).start()

        scatter(j + 1, 1)
        return carry

    lax.fori_loop(0, KC // 2, step, 0)
    if KC % 2:
        scatter(KC - 1, 0)

    plsc.subcore_barrier()

    @pl.when(c == 0)
    def _():
        _rowwise(s, lambda r: pltpu.sync_copy(acc.at[r], out0.at[r]))

    @pl.when(c == 1)
    def _():
        _rowwise(s, lambda r: pltpu.sync_copy(acc.at[r], out1.at[r]))


_agg_plain = pl.kernel(
    _agg_body,
    out_type=[jax.ShapeDtypeStruct((N, D), jnp.float32),
              jax.ShapeDtypeStruct((N, D), jnp.float32)],
    mesh=plsc.VectorSubcoreMesh(core_axis_name="c", subcore_axis_name="s"),
    scratch_types=[
        pltpu.VMEM_SHARED((N, D), jnp.float32),
        pltpu.VMEM((EPT,), jnp.int32),
        pltpu.VMEM((KC, CH), jnp.int32),
        pltpu.VMEM((CH, D), jnp.float32),
        pltpu.VMEM((CH, D), jnp.float32),
        pltpu.SemaphoreType.DMA,
        pltpu.SemaphoreType.DMA,
    ])


_L = 16
_NV = EPT // _L
_NPAD = 10112
_CB = 640


def _count_body(dst1d, zerosp, cnt0, cnt1,
                cnt_all, dst_flat, cnt_local, slab, out_stage):
    c = lax.axis_index("c")
    s = lax.axis_index("s")

    pltpu.sync_copy(dst1d.at[c * NS + s], dst_flat)
    pltpu.sync_copy(zerosp, cnt_local)
    ones_v = jnp.ones((_L,), jnp.float32)

    def cstep(i, carry):
        idx = dst_flat[pl.ds(i * _L, _L)]
        plsc.addupdate_scatter(cnt_local, [idx], ones_v)
        return carry

    lax.fori_loop(0, _NV, cstep, 0)

    pltpu.sync_copy(cnt_local, cnt_all.at[s])
    plsc.subcore_barrier()

    def reduce_range(ncols):
        pltpu.sync_copy(cnt_all.at[:, pl.ds(s * _CB, ncols)],
                        slab.at[:, pl.ds(0, ncols)])

        col0 = jnp.zeros((_L,), jnp.int32)
        lanes = lax.iota(jnp.int32, _L)

        def rstep(b, carry):
            tot = slab[0, pl.ds(b * _L, _L)]
            for t in range(1, NS):
                tot = tot + slab[t, pl.ds(b * _L, _L)]
            plsc.store_scatter(out_stage, [b * _L + lanes, col0], tot)
            return carry

        lax.fori_loop(0, ncols // _L, rstep, 0)

    @pl.when(s < NS - 1)
    def _():
        reduce_range(_CB)

    @pl.when(s == NS - 1)
    def _():
        reduce_range(_NPAD - (NS - 1) * _CB)

    def write(o):
        @pl.when(s < NS - 1)
        def _():
            pltpu.sync_copy(out_stage.at[pl.ds(0, _CB)],
                            o.at[pl.ds(s * _CB, _CB)])

        @pl.when(s == NS - 1)
        def _():
            pltpu.sync_copy(out_stage.at[pl.ds(0, N - (NS - 1) * _CB)],
                            o.at[pl.ds((NS - 1) * _CB, N - (NS - 1) * _CB)])

    @pl.when(c == 0)
    def _():
        write(cnt0)

    @pl.when(c == 1)
    def _():
        write(cnt1)


_count = pl.kernel(
    _count_body,
    out_type=[jax.ShapeDtypeStruct((N, _L), jnp.float32),
              jax.ShapeDtypeStruct((N, _L), jnp.float32)],
    mesh=plsc.VectorSubcoreMesh(core_axis_name="c", subcore_axis_name="s"),
    compiler_params=pltpu.CompilerParams(needs_layout_passes=False),
    scratch_types=[
        pltpu.VMEM_SHARED((NS, _NPAD), jnp.float32),
        pltpu.VMEM((EPT,), jnp.int32),
        pltpu.VMEM((_NPAD,), jnp.float32),
        pltpu.VMEM((NS, _CB), jnp.float32),
        pltpu.VMEM((_CB, _L), jnp.float32),
    ])


_BLK = 1000
_row_spec = pl.BlockSpec((_BLK, D), lambda i: (i, 0))
_cnt_spec = pl.BlockSpec((_BLK, _L), lambda i: (i, 0))
_w_spec = pl.BlockSpec((D, D), lambda i: (0, 0))
_b_spec = pl.BlockSpec((1, D), lambda i: (0, 0))


def _densew_body(h, wr, b, o):
    o[...] = (jnp.dot(h[...], wr[...], preferred_element_type=jnp.float32)
              + b[...])


def _densew(h, wrt, b2d):
    return pl.pallas_call(
        _densew_body,
        grid=(N // _BLK,),
        in_specs=[_row_spec, _w_spec, _b_spec],
        out_specs=_row_spec,
        out_shape=jax.ShapeDtypeStruct((N, D), jnp.float32),
    )(h, wrt, b2d)


def _densec_body(relu, p0, p1, c0, c1, hr, wl, o):
    cnt = c0[:, 0:1] + c1[:, 0:1]
    inv = 1.0 / jnp.maximum(cnt, 1.0)
    agg = (p0[...] + p1[...]) * inv
    out = jnp.dot(agg, wl[...], preferred_element_type=jnp.float32) + hr[...]
    o[...] = jnp.maximum(out, 0.0) if relu else out


def _densec(p0, p1, c0, c1, hr, wlt, relu):
    return pl.pallas_call(
        functools.partial(_densec_body, relu),
        grid=(N // _BLK,),
        in_specs=[_row_spec, _row_spec, _cnt_spec, _cnt_spec, _row_spec,
                  _w_spec],
        out_specs=_row_spec,
        out_shape=jax.ShapeDtypeStruct((N, D), jnp.float32),
    )(p0, p1, c0, c1, hr, wlt)


def kernel(x, e, Wl1, Wr1, b1, Wl2, Wr2, b2, Wl3, Wr3, b3):
    src2d = e[0].reshape(NC * NS, EPT)
    dst2d = e[1].reshape(NC * NS, KC, CH)
    dst1d = e[1].reshape(NC * NS, EPT)
    zeros = jnp.zeros((N, D), jnp.float32)
    zerosp = jnp.zeros((_NPAD,), jnp.float32)

    hr1 = _densew(x, Wr1.T, b1.reshape(1, D))
    c0, c1 = _count(dst1d, zerosp)
    p0, p1 = _agg_plain(x, src2d, dst2d, zeros)
    h1 = _densec(p0, p1, c0, c1, hr1, Wl1.T, True)
    hr2 = _densew(h1, Wr2.T, b2.reshape(1, D))
    p0, p1 = _agg_plain(h1, src2d, dst2d, zeros)
    h2 = _densec(p0, p1, c0, c1, hr2, Wl2.T, True)
    hr3 = _densew(h2, Wr3.T, b3.reshape(1, D))
    p0, p1 = _agg_plain(h2, src2d, dst2d, zeros)
    return _densec(p0, p1, c0, c1, hr3, Wl3.T, False)
